# baseline (speedup 1.0000x reference)
import functools
import math

import jax
import jax.numpy as jnp
from jax import lax
from jax.experimental import pallas as pl
from jax.experimental.pallas import tpu as pltpu
from jax.experimental.pallas import tpu_sc as plsc

_SIZE = 1000
_SMOOTH = 0.1
_CONF = 1.0 - _SMOOTH
_EPS = _SMOOTH / (_SIZE - 2)
_C = _CONF * math.log(_CONF) + (_SIZE - 2) * _EPS * math.log(_EPS)

_N = 16384
_N_SC = 10240
_N_TC = _N - _N_SC
_BM = 1024

_NW = 32
_RW = _N_SC // _NW
_GR = 16
_NG = _RW // _GR
_GE = _GR * _SIZE


def _sc_body(x_hbm, t_hbm, s_out, p_out, xb0, xb1, tb0, tb1, sbuf, pbuf,
             sx0, sx1, st0, st1):
    wid = lax.axis_index("s") * 2 + lax.axis_index("c")
    base = wid * _RW * _SIZE

    def grp_start(g):
        return base + g * _GE

    pltpu.async_copy(x_hbm.at[pl.ds(grp_start(0), _GE)], xb0, sx0)
    pltpu.async_copy(t_hbm.at[pl.ds(grp_start(0), _GE)], tb0, st0)
    pltpu.async_copy(x_hbm.at[pl.ds(grp_start(1), _GE)], xb1, sx1)
    pltpu.async_copy(t_hbm.at[pl.ds(grp_start(1), _GE)], tb1, st1)

    idx_base = lax.iota(jnp.int32, 16) * _SIZE
    neg_inf = jnp.full((16,), -jnp.inf, dtype=jnp.float32)
    zeros = jnp.zeros((16,), dtype=jnp.float32)

    def process(xb, tb, g, pacc):
        def body(j, carry):
            idx, accs = carry
            for k in range(8):
                idx_k = idx + k
                xc = plsc.load_gather(xb, [idx_k])
                tc = plsc.load_gather(tb, [idx_k])
                tmax, xsel, sumx, sexp = accs[k % 4]
                better = tc > tmax
                tmax = jnp.maximum(tmax, tc)
                xsel = jnp.where(better, xc, xsel)
                sumx = sumx + xc
                sexp = sexp + jnp.exp(xc)
                accs[k % 4] = (tmax, xsel, sumx, sexp)
            return (idx + 8, accs)

        init = (idx_base, [(neg_inf, zeros, zeros, zeros)] * 4)
        _, accs = lax.fori_loop(0, _SIZE // 8, body, init)

        def merge(a, b):
            ta, xa, sa, ea = a
            tb_, xb_, sb_, eb_ = b
            bet = tb_ > ta
            return (jnp.maximum(ta, tb_), jnp.where(bet, xb_, xa),
                    sa + sb_, ea + eb_)

        tmax, xsel, sumx, sexp = merge(merge(accs[0], accs[1]),
                                       merge(accs[2], accs[3]))

        x0 = plsc.load_gather(xb, [idx_base])
        t0 = plsc.load_gather(tb, [idx_base])
        pad = t0 >= tmax
        lin = _C - _CONF * xsel - _EPS * (sumx - x0 - xsel)
        pacc = pacc + jnp.where(pad, 0.0, lin)
        s_vec = jnp.where(pad, 1.0, sexp)
        sbuf[pl.ds(g * _GR, _GR)] = s_vec
        return pacc

    def outer(i, pacc):
        for b in range(2):
            g = i * 2 + b
            xb = xb0 if b == 0 else xb1
            tb = tb0 if b == 0 else tb1
            sx = sx0 if b == 0 else sx1
            st = st0 if b == 0 else st1
            pltpu.make_async_copy(x_hbm.at[pl.ds(0, _GE)], xb, sx).wait()
            pltpu.make_async_copy(t_hbm.at[pl.ds(0, _GE)], tb, st).wait()
            pacc = process(xb, tb, g, pacc)

            @pl.when(g + 2 < _NG)
            def _():
                nxt = grp_start(g + 2)
                pltpu.async_copy(x_hbm.at[pl.ds(nxt, _GE)], xb, sx)
                pltpu.async_copy(t_hbm.at[pl.ds(nxt, _GE)], tb, st)

        return pacc

    pacc = lax.fori_loop(0, _NG // 2, outer, zeros)
    pbuf[...] = pacc
    pltpu.sync_copy(sbuf, s_out.at[pl.ds(wid * _RW, _RW)])
    pltpu.sync_copy(pbuf, p_out.at[pl.ds(wid * 16, 16)])


def _tc_body(x_ref, t_ref, out_ref):
    i = pl.program_id(0)
    x = x_ref[...]
    t = t_ref[...]

    sexp = jnp.sum(jnp.exp(x), axis=1)
    lse = jnp.log(sexp)
    sumx = jnp.sum(x, axis=1)
    x0 = x[:, 0]

    tmax = jnp.max(t, axis=1, keepdims=True)
    hit = t == tmax
    xt = jnp.sum(jnp.where(hit, x, 0.0), axis=1)
    pad = hit[:, 0]

    lin = _C - _CONF * xt - _EPS * (sumx - x0 - xt)
    kl = jnp.where(pad, 0.0, lse + lin)
    part = jnp.sum(kl)

    @pl.when(i == 0)
    def _():
        out_ref[0, 0] = 0.0

    out_ref[0, 0] += part


def _finish_body(s_ref, p_ref, tc_ref, out_ref):
    total = jnp.sum(jnp.log(s_ref[...])) + jnp.sum(p_ref[...])
    out_ref[0, 0] = total + tc_ref[0, 0]


@jax.jit
def kernel(x, target):
    x2 = x.reshape(-1, _SIZE)
    t2 = target.reshape(-1, _SIZE)
    xs = x2[_N_TC:].reshape(-1)
    ts = t2[_N_TC:].reshape(-1)

    mesh = plsc.VectorSubcoreMesh(core_axis_name="c", subcore_axis_name="s")
    sc = functools.partial(
        pl.kernel,
        mesh=mesh,
        compiler_params=pltpu.CompilerParams(needs_layout_passes=False),
        out_type=[
            jax.ShapeDtypeStruct((_N_SC,), jnp.float32),
            jax.ShapeDtypeStruct((_NW * 16,), jnp.float32),
        ],
        scratch_types=[
            pltpu.VMEM((_GE,), jnp.float32),
            pltpu.VMEM((_GE,), jnp.float32),
            pltpu.VMEM((_GE,), jnp.float32),
            pltpu.VMEM((_GE,), jnp.float32),
            pltpu.VMEM((_RW,), jnp.float32),
            pltpu.VMEM((16,), jnp.float32),
            pltpu.SemaphoreType.DMA,
            pltpu.SemaphoreType.DMA,
            pltpu.SemaphoreType.DMA,
            pltpu.SemaphoreType.DMA,
        ],
    )(_sc_body)
    s, p = sc(xs, ts)

    tc_part = pl.pallas_call(
        _tc_body,
        grid=(_N_TC // _BM,),
        in_specs=[
            pl.BlockSpec((_BM, _SIZE), lambda i: (i, 0)),
            pl.BlockSpec((_BM, _SIZE), lambda i: (i, 0)),
        ],
        out_specs=pl.BlockSpec(
            (1, 1), lambda i: (0, 0), memory_space=pltpu.SMEM
        ),
        out_shape=jax.ShapeDtypeStruct((1, 1), jnp.float32),
    )(x2, t2)

    out = pl.pallas_call(
        _finish_body,
        in_specs=[
            pl.BlockSpec((80, 128), lambda: (0, 0)),
            pl.BlockSpec((4, 128), lambda: (0, 0)),
            pl.BlockSpec((1, 1), lambda: (0, 0), memory_space=pltpu.SMEM),
        ],
        out_specs=pl.BlockSpec((1, 1), lambda: (0, 0), memory_space=pltpu.SMEM),
        out_shape=jax.ShapeDtypeStruct((1, 1), jnp.float32),
    )(s.reshape(80, 128), p.reshape(4, 128), tc_part)
    return out[0, 0] / _N


# speedup vs baseline: 1.4362x; 1.4362x over previous
import functools
import math

import jax
import jax.numpy as jnp
from jax import lax
from jax.experimental import pallas as pl
from jax.experimental.pallas import tpu as pltpu
from jax.experimental.pallas import tpu_sc as plsc

_SIZE = 1000
_SMOOTH = 0.1
_CONF = 1.0 - _SMOOTH
_EPS = _SMOOTH / (_SIZE - 2)
_C = _CONF * math.log(_CONF) + (_SIZE - 2) * _EPS * math.log(_EPS)

_N = 16384
_N_SC = 4096
_N_TC = _N - _N_SC
_BM = 1024

_NW = 32
_RW = _N_SC // _NW
_GR = 16
_NG = _RW // _GR
_GE = _GR * _SIZE


def _sc_body(x_hbm, t_hbm, s_out, p_out, xb0, xb1, tb0, tb1, sbuf, pbuf,
             sx0, sx1, st0, st1):
    wid = lax.axis_index("s") * 2 + lax.axis_index("c")
    base = _N_TC + wid * _RW

    def grp_start(g):
        return base + g * _GR

    pltpu.async_copy(x_hbm.at[pl.ds(grp_start(0), _GR), :], xb0, sx0)
    pltpu.async_copy(t_hbm.at[pl.ds(grp_start(0), _GR), :], tb0, st0)
    pltpu.async_copy(x_hbm.at[pl.ds(grp_start(1), _GR), :], xb1, sx1)
    pltpu.async_copy(t_hbm.at[pl.ds(grp_start(1), _GR), :], tb1, st1)

    lane = lax.iota(jnp.int32, 16)
    neg_inf = jnp.full((16,), -jnp.inf, dtype=jnp.float32)
    zeros = jnp.zeros((16,), dtype=jnp.float32)
    izeros = jnp.zeros((16,), dtype=jnp.int32)

    def process(xb, tb, g, pacc):
        def body(j, carry):
            jv, accs = carry
            for k in range(8):
                jv_k = jv + k
                xc = plsc.load_gather(xb, [lane, jv_k])
                tc = plsc.load_gather(tb, [lane, jv_k])
                tmax, xsel, sumx, sexp = accs[k % 4]
                better = tc > tmax
                tmax = jnp.maximum(tmax, tc)
                xsel = jnp.where(better, xc, xsel)
                sumx = sumx + xc
                sexp = sexp + jnp.exp(xc)
                accs[k % 4] = (tmax, xsel, sumx, sexp)
            return (jv + 8, accs)

        init = (izeros, [(neg_inf, zeros, zeros, zeros)] * 4)
        _, accs = lax.fori_loop(0, _SIZE // 8, body, init)

        def merge(a, b):
            ta, xa, sa, ea = a
            tb_, xb_, sb_, eb_ = b
            bet = tb_ > ta
            return (jnp.maximum(ta, tb_), jnp.where(bet, xb_, xa),
                    sa + sb_, ea + eb_)

        tmax, xsel, sumx, sexp = merge(merge(accs[0], accs[1]),
                                       merge(accs[2], accs[3]))

        x0 = plsc.load_gather(xb, [lane, izeros])
        t0 = plsc.load_gather(tb, [lane, izeros])
        pad = t0 >= tmax
        lin = _C - _CONF * xsel - _EPS * (sumx - x0 - xsel)
        pacc = pacc + jnp.where(pad, 0.0, lin)
        s_vec = jnp.where(pad, 1.0, sexp)
        sbuf[pl.ds(g * _GR, _GR)] = s_vec
        return pacc

    def outer(i, pacc):
        for b in range(2):
            g = i * 2 + b
            xb = xb0 if b == 0 else xb1
            tb = tb0 if b == 0 else tb1
            sx = sx0 if b == 0 else sx1
            st = st0 if b == 0 else st1
            pltpu.make_async_copy(x_hbm.at[pl.ds(0, _GR), :], xb, sx).wait()
            pltpu.make_async_copy(t_hbm.at[pl.ds(0, _GR), :], tb, st).wait()
            pacc = process(xb, tb, g, pacc)

            @pl.when(g + 2 < _NG)
            def _():
                nxt = grp_start(g + 2)
                pltpu.async_copy(x_hbm.at[pl.ds(nxt, _GR), :], xb, sx)
                pltpu.async_copy(t_hbm.at[pl.ds(nxt, _GR), :], tb, st)

        return pacc

    pacc = lax.fori_loop(0, _NG // 2, outer, zeros)
    pbuf[...] = pacc
    pltpu.sync_copy(sbuf, s_out.at[pl.ds(wid * _RW, _RW)])
    pltpu.sync_copy(pbuf, p_out.at[pl.ds(wid * 16, 16)])


def _tc_body(x_ref, t_ref, out_ref):
    i = pl.program_id(0)
    x = x_ref[...]
    t = t_ref[...]

    sexp = jnp.sum(jnp.exp(x), axis=1)
    lse = jnp.log(sexp)
    sumx = jnp.sum(x, axis=1)
    x0 = x[:, 0]

    tmax = jnp.max(t, axis=1, keepdims=True)
    hit = t == tmax
    xt = jnp.sum(jnp.where(hit, x, 0.0), axis=1)
    pad = hit[:, 0]

    lin = _C - _CONF * xt - _EPS * (sumx - x0 - xt)
    kl = jnp.where(pad, 0.0, lse + lin)
    part = jnp.sum(kl)

    @pl.when(i == 0)
    def _():
        out_ref[0, 0] = 0.0

    out_ref[0, 0] += part


def _finish_body(s_ref, p_ref, tc_ref, out_ref):
    total = jnp.sum(jnp.log(s_ref[...])) + jnp.sum(p_ref[...])
    out_ref[0, 0] = total + tc_ref[0, 0]


@jax.jit
def kernel(x, target):
    x2 = x.reshape(-1, _SIZE)
    t2 = target.reshape(-1, _SIZE)

    mesh = plsc.VectorSubcoreMesh(core_axis_name="c", subcore_axis_name="s")
    sc = functools.partial(
        pl.kernel,
        mesh=mesh,
        compiler_params=pltpu.CompilerParams(
            needs_layout_passes=False, use_tc_tiling_on_sc=True),
        out_type=[
            jax.ShapeDtypeStruct((_N_SC,), jnp.float32),
            jax.ShapeDtypeStruct((_NW * 16,), jnp.float32),
        ],
        scratch_types=[
            pltpu.VMEM((_GR, _SIZE), jnp.float32),
            pltpu.VMEM((_GR, _SIZE), jnp.float32),
            pltpu.VMEM((_GR, _SIZE), jnp.float32),
            pltpu.VMEM((_GR, _SIZE), jnp.float32),
            pltpu.VMEM((_RW,), jnp.float32),
            pltpu.VMEM((16,), jnp.float32),
            pltpu.SemaphoreType.DMA,
            pltpu.SemaphoreType.DMA,
            pltpu.SemaphoreType.DMA,
            pltpu.SemaphoreType.DMA,
        ],
    )(_sc_body)
    s, p = sc(x2, t2)

    tc_part = pl.pallas_call(
        _tc_body,
        grid=(_N_TC // _BM,),
        in_specs=[
            pl.BlockSpec((_BM, _SIZE), lambda i: (i, 0)),
            pl.BlockSpec((_BM, _SIZE), lambda i: (i, 0)),
        ],
        out_specs=pl.BlockSpec(
            (1, 1), lambda i: (0, 0), memory_space=pltpu.SMEM
        ),
        out_shape=jax.ShapeDtypeStruct((1, 1), jnp.float32),
    )(x2, t2)

    out = pl.pallas_call(
        _finish_body,
        in_specs=[
            pl.BlockSpec((32, 128), lambda: (0, 0)),
            pl.BlockSpec((4, 128), lambda: (0, 0)),
            pl.BlockSpec((1, 1), lambda: (0, 0), memory_space=pltpu.SMEM),
        ],
        out_specs=pl.BlockSpec((1, 1), lambda: (0, 0), memory_space=pltpu.SMEM),
        out_shape=jax.ShapeDtypeStruct((1, 1), jnp.float32),
    )(s.reshape(32, 128), p.reshape(4, 128), tc_part)
    return out[0, 0] / _N
